# baseline (speedup 1.0000x reference)
import functools

import jax
import jax.numpy as jnp
from jax import lax
from jax.experimental import pallas as pl
from jax.experimental.pallas import tpu as pltpu
from jax.experimental.pallas import tpu_sc as plsc

_NUM_SC_CORES = 2
_NUM_SC_SUBCORES = 16
_NUM_WORKERS = _NUM_SC_CORES * _NUM_SC_SUBCORES

_GCHUNK = 128


def _sc_gather(table, idx, width):
    n_idx = idx.shape[0]
    b_per_w = n_idx // _NUM_WORKERS
    assert n_idx % _NUM_WORKERS == 0 and b_per_w % _GCHUNK == 0
    n_chunks = b_per_w // _GCHUNK

    mesh = plsc.VectorSubcoreMesh(core_axis_name="c", subcore_axis_name="s")

    @functools.partial(
        pl.kernel,
        mesh=mesh,
        out_type=jax.ShapeDtypeStruct((n_idx, width), table.dtype),
        scratch_types=[
            pltpu.VMEM((b_per_w,), jnp.int32),
            pltpu.VMEM((_GCHUNK, width), table.dtype),
            pltpu.VMEM((_GCHUNK, width), table.dtype),
            pltpu.SemaphoreType.DMA,
            pltpu.SemaphoreType.DMA,
            pltpu.SemaphoreType.DMA,
            pltpu.SemaphoreType.DMA,
        ],
    )
    def gather_kernel(table_hbm, idx_hbm, out_hbm,
                      idx_v, buf0, buf1, g0, g1, w0, w1):
        wid = lax.axis_index("s") * _NUM_SC_CORES + lax.axis_index("c")
        base = wid * b_per_w
        pltpu.sync_copy(idx_hbm.at[pl.ds(base, b_per_w)], idx_v)

        bufs = (buf0, buf1)
        gsems = (g0, g1)
        wsems = (w0, w1)

        def start_gather(j):
            return pltpu.async_copy(
                table_hbm.at[idx_v.at[pl.ds(j * _GCHUNK, _GCHUNK)]],
                bufs[j % 2], gsems[j % 2])

        def start_writeout(j):
            return pltpu.async_copy(
                bufs[j % 2],
                out_hbm.at[pl.ds(base + j * _GCHUNK, _GCHUNK)],
                wsems[j % 2])

        gathers = [None] * n_chunks
        writes = [None] * n_chunks
        gathers[0] = start_gather(0)
        for j in range(n_chunks):
            gathers[j].wait()
            if j + 1 < n_chunks:
                if j >= 1:
                    writes[j - 1].wait()
                gathers[j + 1] = start_gather(j + 1)
            writes[j] = start_writeout(j)
        if n_chunks >= 2:
            writes[n_chunks - 2].wait()
        writes[n_chunks - 1].wait()

    return gather_kernel(table, idx)


def _mlp_body(x_ref, w1_ref, b1_ref, w2_ref, b2_ref, o_ref):
    x = x_ref[...]
    xe = jax.lax.bitcast_convert_type(x << 16, jnp.float32)
    xo = jax.lax.bitcast_convert_type((x >> 16) << 16, jnp.float32)
    c = jnp.concatenate([xe, xo], axis=-1).astype(jnp.bfloat16)
    w1 = w1_ref[...].astype(jnp.bfloat16)
    dn = (((1,), (0,)), ((), ()))
    h = lax.dot_general(c, w1, dn, preferred_element_type=jnp.float32)
    h = jnp.maximum(h + b1_ref[...], 0.0).astype(jnp.bfloat16)
    w2 = w2_ref[...].astype(jnp.bfloat16)
    o = lax.dot_general(h, w2, dn, preferred_element_type=jnp.float32)
    o_ref[...] = o + b2_ref[...]


def _mlp(packed, w1, b1, w2, b2, interpret=False):
    b, uw = packed.shape
    k, hid = w1.shape
    bm = 1024
    return pl.pallas_call(
        _mlp_body,
        grid=(b // bm,),
        in_specs=[
            pl.BlockSpec((bm, uw), lambda i: (i, 0)),
            pl.BlockSpec((k, hid), lambda i: (0, 0)),
            pl.BlockSpec((1, hid), lambda i: (0, 0)),
            pl.BlockSpec((hid, hid), lambda i: (0, 0)),
            pl.BlockSpec((1, hid), lambda i: (0, 0)),
        ],
        out_specs=pl.BlockSpec((bm, hid), lambda i: (i, 0)),
        out_shape=jax.ShapeDtypeStruct((b, hid), jnp.float32),
        interpret=interpret,
    )(packed, w1, b1.reshape(1, hid), w2, b2.reshape(1, hid))


def kernel(age, gender, disability, age_table, gender_table, disability_table,
           W1, b1, W2, b2):
    emb = age_table.shape[1]
    n_age = age_table.shape[0]
    n_gender = gender_table.shape[0]
    n_dis = disability_table.shape[0]
    n_combo = n_age * n_gender * n_dis
    width = 4 * emb
    uw = width // 2

    def pack(t):
        n = t.shape[0]
        return jax.lax.bitcast_convert_type(
            t.astype(jnp.bfloat16).reshape(n, emb // 2, 2), jnp.uint32)

    age_u, gen_u, dis_u = pack(age_table), pack(gender_table), pack(
        disability_table)
    table_u32 = jnp.concatenate(
        [jnp.broadcast_to(age_u[:, None, None, :],
                          (n_age, n_gender, n_dis, emb // 2)),
         jnp.broadcast_to(gen_u[None, :, None, :],
                          (n_age, n_gender, n_dis, emb // 2)),
         jnp.broadcast_to(dis_u[None, None, :, :],
                          (n_age, n_gender, n_dis, emb // 2)),
         jnp.zeros((n_age, n_gender, n_dis, uw - 3 * emb // 2), jnp.uint32)],
        axis=-1,
    ).reshape(n_combo, uw)

    idx = (age.astype(jnp.int32) * (n_gender * n_dis)
           + gender.astype(jnp.int32) * n_dis + disability.astype(jnp.int32))

    rows_u32 = _sc_gather(table_u32, idx, uw)

    hid = W1.shape[1]
    w1p = jnp.concatenate(
        [W1, jnp.zeros((width - 3 * emb, hid), W1.dtype)], axis=0)
    w1pp = jnp.concatenate([w1p[0::2], w1p[1::2]], axis=0)
    return _mlp(rows_u32, w1pp, b1, W2, b2)


# speedup vs baseline: 1.0074x; 1.0074x over previous
import functools

import jax
import jax.numpy as jnp
from jax import lax
from jax.experimental import pallas as pl
from jax.experimental.pallas import tpu as pltpu
from jax.experimental.pallas import tpu_sc as plsc

_NUM_SC_CORES = 2
_NUM_SC_SUBCORES = 16
_NUM_WORKERS = _NUM_SC_CORES * _NUM_SC_SUBCORES

_GCHUNK = 128


def _sc_gather(table, idx, width):
    n_idx = idx.shape[0]
    b_per_w = n_idx // _NUM_WORKERS
    assert n_idx % _NUM_WORKERS == 0 and b_per_w % _GCHUNK == 0
    n_chunks = b_per_w // _GCHUNK

    mesh = plsc.VectorSubcoreMesh(core_axis_name="c", subcore_axis_name="s")

    @functools.partial(
        pl.kernel,
        mesh=mesh,
        out_type=jax.ShapeDtypeStruct((n_idx, width), table.dtype),
        scratch_types=[
            pltpu.VMEM((b_per_w,), jnp.int32),
            pltpu.VMEM((_GCHUNK, width), table.dtype),
            pltpu.VMEM((_GCHUNK, width), table.dtype),
            pltpu.SemaphoreType.DMA,
            pltpu.SemaphoreType.DMA,
            pltpu.SemaphoreType.DMA,
            pltpu.SemaphoreType.DMA,
        ],
    )
    def gather_kernel(table_hbm, idx_hbm, out_hbm,
                      idx_v, buf0, buf1, g0, g1, w0, w1):
        wid = lax.axis_index("s") * _NUM_SC_CORES + lax.axis_index("c")
        base = wid * b_per_w
        pltpu.sync_copy(idx_hbm.at[pl.ds(base, b_per_w)], idx_v)

        bufs = (buf0, buf1)
        gsems = (g0, g1)
        wsems = (w0, w1)

        def start_gather(j):
            return pltpu.async_copy(
                table_hbm.at[idx_v.at[pl.ds(j * _GCHUNK, _GCHUNK)]],
                bufs[j % 2], gsems[j % 2])

        def start_writeout(j):
            return pltpu.async_copy(
                bufs[j % 2],
                out_hbm.at[pl.ds(base + j * _GCHUNK, _GCHUNK)],
                wsems[j % 2])

        gathers = [None] * n_chunks
        writes = [None] * n_chunks
        gathers[0] = start_gather(0)
        for j in range(n_chunks):
            gathers[j].wait()
            if j + 1 < n_chunks:
                if j >= 1:
                    writes[j - 1].wait()
                gathers[j + 1] = start_gather(j + 1)
            writes[j] = start_writeout(j)
        if n_chunks >= 2:
            writes[n_chunks - 2].wait()
        writes[n_chunks - 1].wait()

    return gather_kernel(table, idx)


def _mlp_body(x_ref, w1_ref, b1_ref, w2_ref, b2_ref, o_ref):
    x = x_ref[...]
    xe = jax.lax.bitcast_convert_type(x << 16, jnp.float32)
    xo = jax.lax.bitcast_convert_type((x >> 16) << 16, jnp.float32)
    c = jnp.concatenate([xe, xo], axis=-1).astype(jnp.bfloat16)
    w1 = w1_ref[...].astype(jnp.bfloat16)
    dn = (((1,), (0,)), ((), ()))
    h = lax.dot_general(c, w1, dn, preferred_element_type=jnp.float32)
    h = jnp.maximum(h + b1_ref[...], 0.0).astype(jnp.bfloat16)
    w2 = w2_ref[...].astype(jnp.bfloat16)
    o = lax.dot_general(h, w2, dn, preferred_element_type=jnp.float32)
    o_ref[...] = o + b2_ref[...]


def _mlp(packed, w1, b1, w2, b2, interpret=False):
    b, uw = packed.shape
    k, hid = w1.shape
    bm = 2048
    return pl.pallas_call(
        _mlp_body,
        grid=(b // bm,),
        in_specs=[
            pl.BlockSpec((bm, uw), lambda i: (i, 0)),
            pl.BlockSpec((k, hid), lambda i: (0, 0)),
            pl.BlockSpec((1, hid), lambda i: (0, 0)),
            pl.BlockSpec((hid, hid), lambda i: (0, 0)),
            pl.BlockSpec((1, hid), lambda i: (0, 0)),
        ],
        out_specs=pl.BlockSpec((bm, hid), lambda i: (i, 0)),
        out_shape=jax.ShapeDtypeStruct((b, hid), jnp.float32),
        interpret=interpret,
    )(packed, w1, b1.reshape(1, hid), w2, b2.reshape(1, hid))


def kernel(age, gender, disability, age_table, gender_table, disability_table,
           W1, b1, W2, b2):
    emb = age_table.shape[1]
    n_age = age_table.shape[0]
    n_gender = gender_table.shape[0]
    n_dis = disability_table.shape[0]
    n_combo = n_age * n_gender * n_dis
    width = 4 * emb
    uw = width // 2

    def pack(t):
        n = t.shape[0]
        return jax.lax.bitcast_convert_type(
            t.astype(jnp.bfloat16).reshape(n, emb // 2, 2), jnp.uint32)

    age_u, gen_u, dis_u = pack(age_table), pack(gender_table), pack(
        disability_table)
    table_u32 = jnp.concatenate(
        [jnp.broadcast_to(age_u[:, None, None, :],
                          (n_age, n_gender, n_dis, emb // 2)),
         jnp.broadcast_to(gen_u[None, :, None, :],
                          (n_age, n_gender, n_dis, emb // 2)),
         jnp.broadcast_to(dis_u[None, None, :, :],
                          (n_age, n_gender, n_dis, emb // 2)),
         jnp.zeros((n_age, n_gender, n_dis, uw - 3 * emb // 2), jnp.uint32)],
        axis=-1,
    ).reshape(n_combo, uw)

    idx = (age.astype(jnp.int32) * (n_gender * n_dis)
           + gender.astype(jnp.int32) * n_dis + disability.astype(jnp.int32))

    rows_u32 = _sc_gather(table_u32, idx, uw)

    hid = W1.shape[1]
    w1p = jnp.concatenate(
        [W1, jnp.zeros((width - 3 * emb, hid), W1.dtype)], axis=0)
    w1pp = jnp.concatenate([w1p[0::2], w1p[1::2]], axis=0)
    return _mlp(rows_u32, w1pp, b1, W2, b2)
